# baseline (speedup 1.0000x reference)
import functools

import jax
import jax.numpy as jnp
from jax import lax
from jax.experimental import pallas as pl
from jax.experimental.pallas import tpu as pltpu
from jax.experimental.pallas import tpu_sc as plsc

N = 10000
NPAD = 10240
D = 128
HID = 64
E = 160000
NW = 32
EPT = 5376
BB = 128
NBATCH = 40
NB2 = EPT // BB


def _dense_body(xr, xg, xp, Wsr, Wsg, Wsp, asr, asg, asp,
                Wdr, Wdg, Wdp, adr, adg, adp,
                hsr_o, hsg_o, hsp_o, esr_o, esg_o, esp_o, edr_o, edg_o, edp_o):
    hs_r = xr[...] @ Wsr[...]
    hs_g = xg[...] @ Wsg[...]
    hs_p = xp[...] @ Wsp[...]
    hsr_o[...] = hs_r
    hsg_o[...] = hs_g
    hsp_o[...] = hs_p
    esr_o[...] = hs_r @ asr[...]
    esg_o[...] = hs_g @ asg[...]
    esp_o[...] = hs_p @ asp[...]
    xpb = xp[...]
    edr_o[...] = xpb @ (Wdr[...] @ adr[...])
    edg_o[...] = xpb @ (Wdg[...] @ adg[...])
    edp_o[...] = xpb @ (Wdp[...] @ adp[...])


def _dense_stage(x_r, x_g, x_p, Ws, As, Wd, Ad):
    blk = 1000
    grid = (N // blk,)
    row = lambda i: (i, 0)
    full = lambda i: (0, 0)
    outs = pl.pallas_call(
        _dense_body,
        grid=grid,
        in_specs=[pl.BlockSpec((blk, D), row)] * 3
                 + [pl.BlockSpec((D, HID), full)] * 3
                 + [pl.BlockSpec((HID, 1), full)] * 3
                 + [pl.BlockSpec((D, HID), full)] * 3
                 + [pl.BlockSpec((HID, 1), full)] * 3,
        out_specs=[pl.BlockSpec((blk, HID), row)] * 3
                  + [pl.BlockSpec((blk, 1), row)] * 6,
        out_shape=[jax.ShapeDtypeStruct((N, HID), jnp.float32)] * 3
                  + [jax.ShapeDtypeStruct((N, 1), jnp.float32)] * 6,
    )(x_r, x_g, x_p, *Ws, *As, *Wd, *Ad)
    return outs


def _sc_edge_body(s1, d1, e1, f1, h1, s2, d2, e2, f2, h2, s3, d3, e3, f3, h3,
                  z2_hbm, z1_hbm,
                  num_out, den_out,
                  srcv, dstv, esv, edv, srcvA, dstvA, exvA,
                  srcvB, dstvB, exvB, rowsA, rowsB,
                  num_sh, den_sh, semA, semB):
    c = lax.axis_index("c")
    s = lax.axis_index("s")
    w = c * 16 + s
    seg = pl.ds(s * 640, 640)

    def one_type(t, src_hbm, dst_hbm, es_hbm, ed_hbm, hs_hbm):
        pltpu.sync_copy(z2_hbm.at[seg], num_sh.at[seg])
        pltpu.sync_copy(z1_hbm.at[seg], den_sh.at[seg])
        pltpu.sync_copy(es_hbm, esv)
        pltpu.sync_copy(ed_hbm, edv)
        pltpu.sync_copy(src_hbm.at[w], srcv)
        pltpu.sync_copy(dst_hbm.at[w], dstv)
        plsc.subcore_barrier()

        def groups(b, srcb, dstb, exb):
            def group(j, carry2):
                o = b * BB + j * 16
                s16 = srcv[pl.ds(o, 16)]
                d16 = dstv[pl.ds(o, 16)]
                e = plsc.load_gather(esv, [s16]) + plsc.load_gather(edv, [d16])
                e = jnp.where(e > 0, e, 0.2 * e)
                srcb[pl.ds(j * 16, 16)] = s16
                dstb[pl.ds(j * 16, 16)] = jnp.where(s16 == d16, N, d16)
                exb[pl.ds(j * 16, 16)] = jnp.exp(e)
                return carry2

            lax.fori_loop(0, BB // 16, group, 0, unroll=True)

        def batch(b, carry):
            groups(b, srcvA, dstvA, exvA)
            pltpu.async_copy(hs_hbm.at[srcvA], rowsA, semA).wait()

            def scale(g, carry2):
                ex16 = exvA[pl.ds(g * 16, 16)]
                for k in range(16):
                    v = ex16[k]
                    i = g * 16 + k
                    for q in range(HID // 16):
                        rowsA[i, pl.ds(q * 16, 16)] = rowsA[i, pl.ds(q * 16, 16)] * v
                return carry2

            lax.fori_loop(0, BB // 16, scale, 0)
            den_cp = pltpu.async_copy(exvA, den_sh.at[dstvA], sem=semB, add=True)
            num_cp = pltpu.async_copy(rowsA, num_sh.at[dstvA], sem=semA, add=True)
            den_cp.wait()
            num_cp.wait()
            return carry

        lax.fori_loop(0, NBATCH, batch, 0)
        plsc.subcore_barrier()
        pltpu.sync_copy(num_sh.at[seg], num_out.at[t, c, seg])
        pltpu.sync_copy(den_sh.at[seg], den_out.at[t, c, seg])

    one_type(0, s1, d1, e1, f1, h1)
    one_type(1, s2, d2, e2, f2, h2)
    one_type(2, s3, d3, e3, f3, h3)


@functools.lru_cache(maxsize=1)
def _sc_edge_stage():
    return pl.kernel(
        _sc_edge_body,
        out_type=[jax.ShapeDtypeStruct((3, 2, NPAD, HID), jnp.float32),
                  jax.ShapeDtypeStruct((3, 2, NPAD), jnp.float32)],
        mesh=plsc.VectorSubcoreMesh(core_axis_name="c", subcore_axis_name="s"),
        compiler_params=pltpu.CompilerParams(needs_layout_passes=False,
                                             use_tc_tiling_on_sc=False),
        scratch_types=[
            pltpu.VMEM((EPT,), jnp.int32),
            pltpu.VMEM((EPT,), jnp.int32),
            pltpu.VMEM((NPAD,), jnp.float32),
            pltpu.VMEM((NPAD,), jnp.float32),
            pltpu.VMEM((BB,), jnp.int32),
            pltpu.VMEM((BB,), jnp.int32),
            pltpu.VMEM((BB,), jnp.float32),
            pltpu.VMEM((BB,), jnp.int32),
            pltpu.VMEM((BB,), jnp.int32),
            pltpu.VMEM((BB,), jnp.float32),
            pltpu.VMEM((BB, HID), jnp.float32),
            pltpu.VMEM((BB, HID), jnp.float32),
            pltpu.VMEM_SHARED((NPAD, HID), jnp.float32),
            pltpu.VMEM_SHARED((NPAD,), jnp.float32),
            pltpu.SemaphoreType.DMA,
            pltpu.SemaphoreType.DMA,
        ],
    )


def _combine_body(nu, de, h1, h2, h3,
                  es1, ed1, es2, ed2, es3, ed3, b1, b2, b3, Wl, bl, o_ref):
    acc = jnp.zeros(o_ref.shape[:1] + (HID,), jnp.float32)
    nb = nu[...]
    db = de[...]
    for t, (hs, es, ed, b) in enumerate(((h1, es1, ed1, b1),
                                         (h2, es2, ed2, b2),
                                         (h3, es3, ed3, b3))):
        el = es[...] + ed[...]
        el = jnp.where(el > 0, el, 0.2 * el)
        exl = jnp.exp(el)
        num = nb[t, 0] + nb[t, 1] + exl * hs[...]
        den = db[t, 0] + db[t, 1] + exl
        acc = acc + num / den + b[...]
    h = jnp.where(acc > 0, acc, jnp.exp(jnp.minimum(acc, 0.0)) - 1.0)
    o_ref[...] = h @ Wl[...] + bl[...]


def _combine_stage(num, den, hss, ess, eds, bs, W_lin, b_lin):
    blk = 1000
    OUT = W_lin.shape[1]
    grid = (N // blk,)
    row = lambda i: (i, 0)
    args = [num, den]
    specs = [pl.BlockSpec((3, 2, blk, HID), lambda i: (0, 0, i, 0)),
             pl.BlockSpec((3, 2, blk, 1), lambda i: (0, 0, i, 0))]
    args += list(hss)
    specs += [pl.BlockSpec((blk, HID), row)] * 3
    for t in range(3):
        args += [ess[t], eds[t]]
        specs += [pl.BlockSpec((blk, 1), row)] * 2
    args += [b[None, :] for b in bs] + [W_lin, b_lin[None, :]]
    specs += [pl.BlockSpec((1, HID), lambda i: (0, 0))] * 3
    specs += [pl.BlockSpec((HID, OUT), lambda i: (0, 0)),
              pl.BlockSpec((1, OUT), lambda i: (0, 0))]
    return pl.pallas_call(
        _combine_body,
        grid=grid,
        in_specs=specs,
        out_specs=pl.BlockSpec((blk, OUT), row),
        out_shape=jax.ShapeDtypeStruct((N, OUT), jnp.float32),
    )(*args)


def _prep_edges(ei):
    src = ei[0].reshape(NW, E // NW)
    dst = ei[1].reshape(NW, E // NW)
    padw = EPT - E // NW
    src = jnp.pad(src, ((0, 0), (0, padw)))
    dst = jnp.pad(dst, ((0, 0), (0, padw)), constant_values=N)
    return src, dst


def kernel(x_radiomic, x_gene, x_patient, ei_rp, ei_gp, ei_pp,
           W_src_rp, W_dst_rp, a_src_rp, a_dst_rp, b_rp,
           W_src_gp, W_dst_gp, a_src_gp, a_dst_gp, b_gp,
           W_src_pp, W_dst_pp, a_src_pp, a_dst_pp, b_pp,
           W_lin, b_lin):
    As = (a_src_rp[:, None], a_src_gp[:, None], a_src_pp[:, None])
    Ad = (a_dst_rp[:, None], a_dst_gp[:, None], a_dst_pp[:, None])
    (hs_r, hs_g, hs_p, es_r, es_g, es_p, ed_r, ed_g, ed_p) = _dense_stage(
        x_radiomic, x_gene, x_patient,
        (W_src_rp, W_src_gp, W_src_pp), As,
        (W_dst_rp, W_dst_gp, W_dst_pp), Ad)

    z2 = jnp.zeros((NPAD, HID), jnp.float32)
    z1 = jnp.zeros((NPAD,), jnp.float32)
    pad1 = lambda v: jnp.pad(v[:, 0], (0, NPAD - N))

    args = []
    for ei, es, ed, hs in ((ei_rp, es_r, ed_r, hs_r),
                           (ei_gp, es_g, ed_g, hs_g),
                           (ei_pp, es_p, ed_p, hs_p)):
        srcp, dstp = _prep_edges(ei)
        args += [srcp, dstp, pad1(es), pad1(ed), hs]
    num, den = _sc_edge_stage()(*args, z2, z1)

    return _combine_stage(num, den[..., None], (hs_r, hs_g, hs_p),
                          (es_r, es_g, es_p), (ed_r, ed_g, ed_p),
                          (b_rp, b_gp, b_pp), W_lin, b_lin)


# speedup vs baseline: 1.0611x; 1.0611x over previous
import functools

import jax
import jax.numpy as jnp
from jax import lax
from jax.experimental import pallas as pl
from jax.experimental.pallas import tpu as pltpu
from jax.experimental.pallas import tpu_sc as plsc

N = 10000
NPAD = 10240
D = 128
HID = 64
E = 160000
NW = 32
EPT = 5376
BB = 128
NBATCH = 40
NB2 = EPT // BB


def _dense_body(xr, xg, xp, Wsr, Wsg, Wsp, asr, asg, asp,
                Wdr, Wdg, Wdp, adr, adg, adp,
                hsr_o, hsg_o, hsp_o, esr_o, esg_o, esp_o, edr_o, edg_o, edp_o):
    hs_r = xr[...] @ Wsr[...]
    hs_g = xg[...] @ Wsg[...]
    hs_p = xp[...] @ Wsp[...]
    hsr_o[...] = hs_r
    hsg_o[...] = hs_g
    hsp_o[...] = hs_p
    esr_o[...] = hs_r @ asr[...]
    esg_o[...] = hs_g @ asg[...]
    esp_o[...] = hs_p @ asp[...]
    xpb = xp[...]
    edr_o[...] = xpb @ (Wdr[...] @ adr[...])
    edg_o[...] = xpb @ (Wdg[...] @ adg[...])
    edp_o[...] = xpb @ (Wdp[...] @ adp[...])


def _dense_stage(x_r, x_g, x_p, Ws, As, Wd, Ad):
    blk = 1000
    grid = (N // blk,)
    row = lambda i: (i, 0)
    full = lambda i: (0, 0)
    outs = pl.pallas_call(
        _dense_body,
        grid=grid,
        in_specs=[pl.BlockSpec((blk, D), row)] * 3
                 + [pl.BlockSpec((D, HID), full)] * 3
                 + [pl.BlockSpec((HID, 1), full)] * 3
                 + [pl.BlockSpec((D, HID), full)] * 3
                 + [pl.BlockSpec((HID, 1), full)] * 3,
        out_specs=[pl.BlockSpec((blk, HID), row)] * 3
                  + [pl.BlockSpec((blk, 1), row)] * 6,
        out_shape=[jax.ShapeDtypeStruct((N, HID), jnp.float32)] * 3
                  + [jax.ShapeDtypeStruct((N, 1), jnp.float32)] * 6,
    )(x_r, x_g, x_p, *Ws, *As, *Wd, *Ad)
    return outs


def _sc_edge_body(src_hbm, dst_hbm, es_hbm, ed_hbm, hs_hbm, z2_hbm, z1_hbm,
                  num_out, den_out,
                  srcv, dstv, esv, edv, srcvA, dstvA, exvA,
                  srcvB, dstvB, exvB, rowsA, rowsB,
                  num_sh, den_sh, semA, semB):
    c = lax.axis_index("c")
    s = lax.axis_index("s")
    w = c * 16 + s
    seg = pl.ds(s * 640, 640)
    pltpu.sync_copy(z2_hbm.at[seg], num_sh.at[seg])
    pltpu.sync_copy(z1_hbm.at[seg], den_sh.at[seg])
    pltpu.sync_copy(es_hbm, esv)
    pltpu.sync_copy(ed_hbm, edv)
    pltpu.sync_copy(src_hbm.at[w], srcv)
    pltpu.sync_copy(dst_hbm.at[w], dstv)

    plsc.subcore_barrier()

    def groups(b, srcb, dstb, exb):
        def group(j, carry2):
            o = b * BB + j * 16
            s16 = srcv[pl.ds(o, 16)]
            d16 = dstv[pl.ds(o, 16)]
            e = plsc.load_gather(esv, [s16]) + plsc.load_gather(edv, [d16])
            e = jnp.where(e > 0, e, 0.2 * e)
            srcb[pl.ds(j * 16, 16)] = s16
            dstb[pl.ds(j * 16, 16)] = jnp.where(s16 == d16, N, d16)
            exb[pl.ds(j * 16, 16)] = jnp.exp(e)
            return carry2

        lax.fori_loop(0, BB // 16, group, 0, unroll=True)

    def batch(b, carry):
        groups(b, srcvA, dstvA, exvA)
        KG = 4
        for q in range(KG):
            pltpu.async_copy(hs_hbm.at[srcvA.at[pl.ds(q * (BB // KG), BB // KG)]],
                             rowsA.at[pl.ds(q * (BB // KG), BB // KG)], semA)
        for q in range(KG):
            pltpu.make_async_copy(hs_hbm.at[srcvA.at[pl.ds(0, BB // KG)]],
                                  rowsA.at[pl.ds(0, BB // KG)], semA).wait()

        def scale(g, carry2):
            ex16 = exvA[pl.ds(g * 16, 16)]
            for k in range(16):
                v = ex16[k]
                i = g * 16 + k
                for q in range(HID // 16):
                    rowsA[i, pl.ds(q * 16, 16)] = rowsA[i, pl.ds(q * 16, 16)] * v
            return carry2

        lax.fori_loop(0, BB // 16, scale, 0)
        den_cp = pltpu.async_copy(exvA, den_sh.at[dstvA], sem=semB, add=True)
        num_cp = pltpu.async_copy(rowsA, num_sh.at[dstvA], sem=semA, add=True)
        den_cp.wait()
        num_cp.wait()
        return carry

    lax.fori_loop(0, NBATCH, batch, 0)
    plsc.subcore_barrier()
    pltpu.sync_copy(num_sh.at[seg], num_out.at[c, seg])
    pltpu.sync_copy(den_sh.at[seg], den_out.at[c, seg])


@functools.lru_cache(maxsize=1)
def _sc_edge_stage():
    return pl.kernel(
        _sc_edge_body,
        out_type=[jax.ShapeDtypeStruct((2, NPAD, HID), jnp.float32),
                  jax.ShapeDtypeStruct((2, NPAD), jnp.float32)],
        mesh=plsc.VectorSubcoreMesh(core_axis_name="c", subcore_axis_name="s"),
        compiler_params=pltpu.CompilerParams(needs_layout_passes=False,
                                             use_tc_tiling_on_sc=False),
        scratch_types=[
            pltpu.VMEM((EPT,), jnp.int32),
            pltpu.VMEM((EPT,), jnp.int32),
            pltpu.VMEM((NPAD,), jnp.float32),
            pltpu.VMEM((NPAD,), jnp.float32),
            pltpu.VMEM((BB,), jnp.int32),
            pltpu.VMEM((BB,), jnp.int32),
            pltpu.VMEM((BB,), jnp.float32),
            pltpu.VMEM((BB,), jnp.int32),
            pltpu.VMEM((BB,), jnp.int32),
            pltpu.VMEM((BB,), jnp.float32),
            pltpu.VMEM((BB, HID), jnp.float32),
            pltpu.VMEM((BB, HID), jnp.float32),
            pltpu.VMEM_SHARED((NPAD, HID), jnp.float32),
            pltpu.VMEM_SHARED((NPAD,), jnp.float32),
            pltpu.SemaphoreType.DMA,
            pltpu.SemaphoreType.DMA,
        ],
    )


def _combine_body(n1, d1, n2, d2, n3, d3, h1, h2, h3,
                  es1, ed1, es2, ed2, es3, ed3, b1, b2, b3, Wl, bl, o_ref):
    acc = jnp.zeros(o_ref.shape[:1] + (HID,), jnp.float32)
    for (nu, de, hs, es, ed, b) in ((n1, d1, h1, es1, ed1, b1),
                                    (n2, d2, h2, es2, ed2, b2),
                                    (n3, d3, h3, es3, ed3, b3)):
        el = es[...] + ed[...]
        el = jnp.where(el > 0, el, 0.2 * el)
        exl = jnp.exp(el)
        nb = nu[...]
        num = nb[0] + nb[1] + exl * hs[...]
        db = de[...]
        den = db[0] + db[1] + exl
        acc = acc + num / den + b[...]
    h = jnp.where(acc > 0, acc, jnp.exp(jnp.minimum(acc, 0.0)) - 1.0)
    o_ref[...] = h @ Wl[...] + bl[...]


def _combine_stage(nums, dens, hss, ess, eds, bs, W_lin, b_lin):
    blk = 1000
    OUT = W_lin.shape[1]
    grid = (N // blk,)
    row = lambda i: (i, 0)
    args = []
    specs = []
    for t in range(3):
        args += [nums[t], dens[t]]
        specs += [pl.BlockSpec((2, blk, HID), lambda i: (0, i, 0)),
                  pl.BlockSpec((2, blk, 1), lambda i: (0, i, 0))]
    args += list(hss)
    specs += [pl.BlockSpec((blk, HID), row)] * 3
    for t in range(3):
        args += [ess[t], eds[t]]
        specs += [pl.BlockSpec((blk, 1), row)] * 2
    args += [b[None, :] for b in bs] + [W_lin, b_lin[None, :]]
    specs += [pl.BlockSpec((1, HID), lambda i: (0, 0))] * 3
    specs += [pl.BlockSpec((HID, OUT), lambda i: (0, 0)),
              pl.BlockSpec((1, OUT), lambda i: (0, 0))]
    return pl.pallas_call(
        _combine_body,
        grid=grid,
        in_specs=specs,
        out_specs=pl.BlockSpec((blk, OUT), row),
        out_shape=jax.ShapeDtypeStruct((N, OUT), jnp.float32),
    )(*args)


def _prep_edges(ei):
    src = ei[0].reshape(NW, E // NW)
    dst = ei[1].reshape(NW, E // NW)
    padw = EPT - E // NW
    src = jnp.pad(src, ((0, 0), (0, padw)))
    dst = jnp.pad(dst, ((0, 0), (0, padw)), constant_values=N)
    return src, dst


def kernel(x_radiomic, x_gene, x_patient, ei_rp, ei_gp, ei_pp,
           W_src_rp, W_dst_rp, a_src_rp, a_dst_rp, b_rp,
           W_src_gp, W_dst_gp, a_src_gp, a_dst_gp, b_gp,
           W_src_pp, W_dst_pp, a_src_pp, a_dst_pp, b_pp,
           W_lin, b_lin):
    As = (a_src_rp[:, None], a_src_gp[:, None], a_src_pp[:, None])
    Ad = (a_dst_rp[:, None], a_dst_gp[:, None], a_dst_pp[:, None])
    (hs_r, hs_g, hs_p, es_r, es_g, es_p, ed_r, ed_g, ed_p) = _dense_stage(
        x_radiomic, x_gene, x_patient,
        (W_src_rp, W_src_gp, W_src_pp), As,
        (W_dst_rp, W_dst_gp, W_dst_pp), Ad)

    z2 = jnp.zeros((NPAD, HID), jnp.float32)
    z1 = jnp.zeros((NPAD,), jnp.float32)
    pad1 = lambda v: jnp.pad(v[:, 0], (0, NPAD - N))

    nums, dens = [], []
    for ei, es, ed, hs in ((ei_rp, es_r, ed_r, hs_r),
                           (ei_gp, es_g, ed_g, hs_g),
                           (ei_pp, es_p, ed_p, hs_p)):
        srcp, dstp = _prep_edges(ei)
        num, den = _sc_edge_stage()(srcp, dstp, pad1(es), pad1(ed), hs, z2, z1)
        nums.append(num)
        dens.append(den[:, :, None])

    return _combine_stage(nums, dens, (hs_r, hs_g, hs_p),
                          (es_r, es_g, es_p), (ed_r, ed_g, ed_p),
                          (b_rp, b_gp, b_pp), W_lin, b_lin)


# speedup vs baseline: 1.0851x; 1.0226x over previous
import functools

import jax
import jax.numpy as jnp
from jax import lax
from jax.experimental import pallas as pl
from jax.experimental.pallas import tpu as pltpu
from jax.experimental.pallas import tpu_sc as plsc

N = 10000
NPAD = 10240
D = 128
HID = 64
E = 160000
NW = 32
EPT = 5376
BB = 128
NBATCH = 40
NB2 = EPT // BB


def _dense_body(xr, xg, xp, Wsr, Wsg, Wsp, asr, asg, asp,
                Wdr, Wdg, Wdp, adr, adg, adp,
                hsr_o, hsg_o, hsp_o, esr_o, esg_o, esp_o, edr_o, edg_o, edp_o):
    hs_r = xr[...] @ Wsr[...]
    hs_g = xg[...] @ Wsg[...]
    hs_p = xp[...] @ Wsp[...]
    hsr_o[...] = hs_r
    hsg_o[...] = hs_g
    hsp_o[...] = hs_p
    esr_o[...] = hs_r @ asr[...]
    esg_o[...] = hs_g @ asg[...]
    esp_o[...] = hs_p @ asp[...]
    xpb = xp[...]
    edr_o[...] = xpb @ (Wdr[...] @ adr[...])
    edg_o[...] = xpb @ (Wdg[...] @ adg[...])
    edp_o[...] = xpb @ (Wdp[...] @ adp[...])


def _dense_stage(x_r, x_g, x_p, Ws, As, Wd, Ad):
    blk = 1000
    grid = (N // blk,)
    row = lambda i: (i, 0)
    full = lambda i: (0, 0)
    outs = pl.pallas_call(
        _dense_body,
        grid=grid,
        in_specs=[pl.BlockSpec((blk, D), row)] * 3
                 + [pl.BlockSpec((D, HID), full)] * 3
                 + [pl.BlockSpec((HID, 1), full)] * 3
                 + [pl.BlockSpec((D, HID), full)] * 3
                 + [pl.BlockSpec((HID, 1), full)] * 3,
        out_specs=[pl.BlockSpec((blk, HID), row)] * 3
                  + [pl.BlockSpec((blk, 1), row)] * 6,
        out_shape=[jax.ShapeDtypeStruct((N, HID), jnp.float32)] * 3
                  + [jax.ShapeDtypeStruct((N, 1), jnp.float32)] * 6,
    )(x_r, x_g, x_p, *Ws, *As, *Wd, *Ad)
    return outs


def _sc_edge_body(src_hbm, dst_hbm, es_hbm, ed_hbm, hs_hbm, z2_hbm, z1_hbm,
                  num_out, den_out,
                  srcv, dstv, esv, edv, srcvA, dstvA, exvA,
                  srcvB, dstvB, exvB, rowsA, rowsB,
                  num_sh, den_sh, semA, semB):
    c = lax.axis_index("c")
    s = lax.axis_index("s")
    w = c * 16 + s
    seg = pl.ds(s * 640, 640)
    pltpu.sync_copy(z2_hbm.at[seg], num_sh.at[seg])
    pltpu.sync_copy(z1_hbm.at[seg], den_sh.at[seg])
    pltpu.sync_copy(es_hbm, esv)
    pltpu.sync_copy(ed_hbm, edv)
    pltpu.sync_copy(src_hbm.at[w], srcv)
    pltpu.sync_copy(dst_hbm.at[w], dstv)

    plsc.subcore_barrier()

    def groups(b, srcb, dstb, exb):
        def group(j, carry2):
            o = b * BB + j * 16
            s16 = srcv[pl.ds(o, 16)]
            d16 = dstv[pl.ds(o, 16)]
            e = plsc.load_gather(esv, [s16]) + plsc.load_gather(edv, [d16])
            e = jnp.where(e > 0, e, 0.2 * e)
            srcb[pl.ds(j * 16, 16)] = s16
            dstb[pl.ds(j * 16, 16)] = jnp.where(s16 == d16, N, d16)
            exb[pl.ds(j * 16, 16)] = jnp.exp(e)
            return carry2

        lax.fori_loop(0, BB // 16, group, 0, unroll=True)

    def batch(b, carry):
        groups(b, srcvA, dstvA, exvA)
        pltpu.async_copy(hs_hbm.at[srcvA], rowsA, semA).wait()

        def scale(g, carry2):
            ex16 = exvA[pl.ds(g * 16, 16)]
            for k in range(16):
                v = ex16[k]
                i = g * 16 + k
                for h in range(HID // 32):
                    xi = plsc.bitcast(rowsA[i, pl.ds(h * 32, 32)], jnp.int32)
                    lo = plsc.bitcast(xi << 16, jnp.float32) * v
                    hi = plsc.bitcast(xi & jnp.int32(-65536), jnp.float32) * v
                    rowsB[i, pl.ds(h * 32, 16)] = lo
                    rowsB[i, pl.ds(h * 32 + 16, 16)] = hi
            return carry2

        lax.fori_loop(0, BB // 16, scale, 0)
        den_cp = pltpu.async_copy(exvA, den_sh.at[dstvA], sem=semB, add=True)
        num_cp = pltpu.async_copy(rowsB, num_sh.at[dstvA], sem=semA, add=True)
        den_cp.wait()
        num_cp.wait()
        return carry

    lax.fori_loop(0, NBATCH, batch, 0)
    plsc.subcore_barrier()
    pltpu.sync_copy(num_sh.at[seg], num_out.at[c, seg])
    pltpu.sync_copy(den_sh.at[seg], den_out.at[c, seg])


@functools.lru_cache(maxsize=1)
def _sc_edge_stage():
    return pl.kernel(
        _sc_edge_body,
        out_type=[jax.ShapeDtypeStruct((2, NPAD, HID), jnp.float32),
                  jax.ShapeDtypeStruct((2, NPAD), jnp.float32)],
        mesh=plsc.VectorSubcoreMesh(core_axis_name="c", subcore_axis_name="s"),
        compiler_params=pltpu.CompilerParams(needs_layout_passes=False,
                                             use_tc_tiling_on_sc=False),
        scratch_types=[
            pltpu.VMEM((EPT,), jnp.int32),
            pltpu.VMEM((EPT,), jnp.int32),
            pltpu.VMEM((NPAD,), jnp.float32),
            pltpu.VMEM((NPAD,), jnp.float32),
            pltpu.VMEM((BB,), jnp.int32),
            pltpu.VMEM((BB,), jnp.int32),
            pltpu.VMEM((BB,), jnp.float32),
            pltpu.VMEM((BB,), jnp.int32),
            pltpu.VMEM((BB,), jnp.int32),
            pltpu.VMEM((BB,), jnp.float32),
            pltpu.VMEM((BB, HID), jnp.bfloat16),
            pltpu.VMEM((BB, HID), jnp.float32),
            pltpu.VMEM_SHARED((NPAD, HID), jnp.float32),
            pltpu.VMEM_SHARED((NPAD,), jnp.float32),
            pltpu.SemaphoreType.DMA,
            pltpu.SemaphoreType.DMA,
        ],
    )


def _combine_body(n1, d1, n2, d2, n3, d3, h1, h2, h3,
                  es1, ed1, es2, ed2, es3, ed3, b1, b2, b3, Wl, bl, o_ref):
    acc = jnp.zeros(o_ref.shape[:1] + (HID,), jnp.float32)
    for (nu, de, hs, es, ed, b) in ((n1, d1, h1, es1, ed1, b1),
                                    (n2, d2, h2, es2, ed2, b2),
                                    (n3, d3, h3, es3, ed3, b3)):
        el = es[...] + ed[...]
        el = jnp.where(el > 0, el, 0.2 * el)
        exl = jnp.exp(el)
        nb = nu[...]
        num = nb[0] + nb[1] + exl * hs[...]
        db = de[...]
        den = db[0] + db[1] + exl
        acc = acc + num / den + b[...]
    h = jnp.where(acc > 0, acc, jnp.exp(jnp.minimum(acc, 0.0)) - 1.0)
    o_ref[...] = h @ Wl[...] + bl[...]


def _combine_stage(nums, dens, hss, ess, eds, bs, W_lin, b_lin):
    blk = 1000
    OUT = W_lin.shape[1]
    grid = (N // blk,)
    row = lambda i: (i, 0)
    args = []
    specs = []
    for t in range(3):
        args += [nums[t], dens[t]]
        specs += [pl.BlockSpec((2, blk, HID), lambda i: (0, i, 0)),
                  pl.BlockSpec((2, blk, 1), lambda i: (0, i, 0))]
    args += list(hss)
    specs += [pl.BlockSpec((blk, HID), row)] * 3
    for t in range(3):
        args += [ess[t], eds[t]]
        specs += [pl.BlockSpec((blk, 1), row)] * 2
    args += [b[None, :] for b in bs] + [W_lin, b_lin[None, :]]
    specs += [pl.BlockSpec((1, HID), lambda i: (0, 0))] * 3
    specs += [pl.BlockSpec((HID, OUT), lambda i: (0, 0)),
              pl.BlockSpec((1, OUT), lambda i: (0, 0))]
    return pl.pallas_call(
        _combine_body,
        grid=grid,
        in_specs=specs,
        out_specs=pl.BlockSpec((blk, OUT), row),
        out_shape=jax.ShapeDtypeStruct((N, OUT), jnp.float32),
    )(*args)


def _prep_edges(ei):
    src = ei[0].reshape(NW, E // NW)
    dst = ei[1].reshape(NW, E // NW)
    padw = EPT - E // NW
    src = jnp.pad(src, ((0, 0), (0, padw)))
    dst = jnp.pad(dst, ((0, 0), (0, padw)), constant_values=N)
    return src, dst


def kernel(x_radiomic, x_gene, x_patient, ei_rp, ei_gp, ei_pp,
           W_src_rp, W_dst_rp, a_src_rp, a_dst_rp, b_rp,
           W_src_gp, W_dst_gp, a_src_gp, a_dst_gp, b_gp,
           W_src_pp, W_dst_pp, a_src_pp, a_dst_pp, b_pp,
           W_lin, b_lin):
    As = (a_src_rp[:, None], a_src_gp[:, None], a_src_pp[:, None])
    Ad = (a_dst_rp[:, None], a_dst_gp[:, None], a_dst_pp[:, None])
    (hs_r, hs_g, hs_p, es_r, es_g, es_p, ed_r, ed_g, ed_p) = _dense_stage(
        x_radiomic, x_gene, x_patient,
        (W_src_rp, W_src_gp, W_src_pp), As,
        (W_dst_rp, W_dst_gp, W_dst_pp), Ad)

    z2 = jnp.zeros((NPAD, HID), jnp.float32)
    z1 = jnp.zeros((NPAD,), jnp.float32)
    pad1 = lambda v: jnp.pad(v[:, 0], (0, NPAD - N))
    perm = jnp.array([c for h in (0, 32) for j in range(16) for c in (h + j, h + j + 16)],
                     dtype=jnp.int32)

    nums, dens = [], []
    for ei, es, ed, hs in ((ei_rp, es_r, ed_r, hs_r),
                           (ei_gp, es_g, ed_g, hs_g),
                           (ei_pp, es_p, ed_p, hs_p)):
        srcp, dstp = _prep_edges(ei)
        hs16 = hs.astype(jnp.bfloat16)[:, perm]
        num, den = _sc_edge_stage()(srcp, dstp, pad1(es), pad1(ed), hs16, z2, z1)
        nums.append(num)
        dens.append(den[:, :, None])

    return _combine_stage(nums, dens, (hs_r, hs_g, hs_p),
                          (es_r, es_g, es_p), (ed_r, ed_g, ed_p),
                          (b_rp, b_gp, b_pp), W_lin, b_lin)
